# speedup vs baseline: 2.9826x; 2.9826x over previous
import functools

import jax
import jax.numpy as jnp
from jax import lax
from jax.experimental import pallas as pl
from jax.experimental.pallas import tpu as pltpu
from jax.experimental.pallas import tpu_sc as plsc

DIM = 128
NC, NS = 2, 16
NW = NC * NS
B = 4096 * 50
BPW = B // NW
CHUNK = 128
NCHUNK = BPW // CHUNK

_mesh = plsc.VectorSubcoreMesh(core_axis_name="c", subcore_axis_name="s")


@functools.partial(
    pl.kernel,
    mesh=_mesh,
    out_type=jax.ShapeDtypeStruct((B, DIM), jnp.float32),
    scratch_types=[
        pltpu.VMEM((NCHUNK, CHUNK), jnp.int32),
        pltpu.VMEM((CHUNK, DIM), jnp.float32),
        pltpu.SemaphoreType.DMA,
    ],
)
def _gather_kernel(table_hbm, idx_hbm, out_hbm, idx_v, rows_v, sem):
    wid = lax.axis_index("s") * NC + lax.axis_index("c")
    base = wid * BPW
    pltpu.sync_copy(idx_hbm.at[wid], idx_v)

    def body(j, carry):
        pltpu.async_copy(table_hbm.at[idx_v.at[j]], rows_v, sem).wait()
        pltpu.sync_copy(rows_v, out_hbm.at[pl.ds(base + j * CHUNK, CHUNK)])
        return carry

    lax.fori_loop(0, NCHUNK, body, 0)


def kernel(token_ids, embeddings):
    idx = jnp.reshape(token_ids.astype(jnp.int32), (NW, NCHUNK, CHUNK))
    out = _gather_kernel(embeddings, idx)
    return jnp.reshape(out, token_ids.shape + (DIM,))


# speedup vs baseline: 3.2808x; 1.1000x over previous
import functools

import jax
import jax.numpy as jnp
from jax import lax
from jax.experimental import pallas as pl
from jax.experimental.pallas import tpu as pltpu
from jax.experimental.pallas import tpu_sc as plsc

DIM = 128
NC, NS = 2, 16
NW = NC * NS
B = 4096 * 50
BPW = B // NW
CHUNK = 128
NCHUNK = BPW // CHUNK
CH = 2 * CHUNK
NG = NCHUNK // 2

_mesh = plsc.VectorSubcoreMesh(core_axis_name="c", subcore_axis_name="s")


@functools.partial(
    pl.kernel,
    mesh=_mesh,
    out_type=jax.ShapeDtypeStruct((B, DIM), jnp.float32),
    scratch_types=[
        pltpu.VMEM((NCHUNK, CHUNK), jnp.int32),
        pltpu.VMEM((CH, DIM), jnp.float32),
        pltpu.VMEM((CH, DIM), jnp.float32),
        pltpu.SemaphoreType.DMA,
        pltpu.SemaphoreType.DMA,
        pltpu.SemaphoreType.DMA,
        pltpu.SemaphoreType.DMA,
    ],
)
def _gather_kernel(table, idx_hbm, out, idx_v, buf_a, buf_b,
                   in_a, in_b, out_a, out_b):
    wid = lax.axis_index("s") * NC + lax.axis_index("c")
    base = wid * BPW
    pltpu.sync_copy(idx_hbm.at[wid], idx_v)

    def gstart(g, buf, sem):
        pltpu.async_copy(table.at[idx_v.at[2 * g]],
                         buf.at[pl.ds(0, CHUNK)], sem)
        pltpu.async_copy(table.at[idx_v.at[2 * g + 1]],
                         buf.at[pl.ds(CHUNK, CHUNK)], sem)

    def gwait(buf, sem):
        pltpu.make_async_copy(table.at[pl.ds(0, CH)], buf, sem).wait()

    def wstart(g, buf, sem):
        pltpu.async_copy(buf, out.at[pl.ds(base + g * CH, CH)], sem)

    def wwait(buf, sem):
        pltpu.make_async_copy(buf, out.at[pl.ds(base, CH)], sem).wait()

    gstart(0, buf_a, in_a)
    gstart(1, buf_b, in_b)

    def body(i, carry):
        g0 = 2 * i
        gwait(buf_a, in_a)
        wstart(g0, buf_a, out_a)
        gwait(buf_b, in_b)
        wstart(g0 + 1, buf_b, out_b)
        wwait(buf_a, out_a)
        gstart(g0 + 2, buf_a, in_a)
        wwait(buf_b, out_b)
        gstart(g0 + 3, buf_b, in_b)
        return carry

    lax.fori_loop(0, (NG - 3) // 2, body, 0)

    gwait(buf_a, in_a)
    wstart(NG - 3, buf_a, out_a)
    gwait(buf_b, in_b)
    wstart(NG - 2, buf_b, out_b)
    wwait(buf_a, out_a)
    gstart(NG - 1, buf_a, in_a)
    gwait(buf_a, in_a)
    wstart(NG - 1, buf_a, out_a)
    wwait(buf_b, out_b)
    wwait(buf_a, out_a)


def kernel(token_ids, embeddings):
    idx = jnp.reshape(token_ids.astype(jnp.int32), (NW, NCHUNK, CHUNK))
    out = _gather_kernel(embeddings, idx)
    return jnp.reshape(out, token_ids.shape + (DIM,))
